# baseline (speedup 1.0000x reference)
import functools

import jax
import jax.numpy as jnp
from jax import lax
from jax.experimental import pallas as pl
from jax.experimental.pallas import tpu as pltpu
from jax.experimental.pallas import tpu_sc as plsc

E = 16
L = 16
D = 2048
B = 4
H = 16
S = 4096
DH = D // H

_NUM_SC_CORES = 2


def _sc_gather(ids16, ep_flat):
    mesh = plsc.ScalarSubcoreMesh(axis_name="core", num_cores=_NUM_SC_CORES)
    rows_per_core = B // _NUM_SC_CORES

    @functools.partial(
        pl.kernel,
        out_type=jax.ShapeDtypeStruct((B, L * D), jnp.float32),
        mesh=mesh,
        scratch_types=[pltpu.SMEM((16,), jnp.int32), pltpu.SemaphoreType.DMA],
    )
    def k(ids_hbm, ep_hbm, out_hbm, ids_smem, sem):
        core = lax.axis_index("core")
        pltpu.async_copy(ids_hbm, ids_smem, sem).wait()
        base = core * rows_per_core
        c0 = pltpu.async_copy(ep_hbm.at[ids_smem[base]], out_hbm.at[base], sem)
        c1 = pltpu.async_copy(
            ep_hbm.at[ids_smem[base + 1]], out_hbm.at[base + 1], sem
        )
        c0.wait()
        c1.wait()

    return k(ids16, ep_flat)


_BN = 256


def _proj_body(x_ref, wk_ref, wv_ref, g_ref, k_ref, v_ref, sg_ref):
    x = x_ref[...]
    k_ref[...] = jnp.dot(x, wk_ref[...], preferred_element_type=jnp.float32)
    v_ref[...] = jnp.dot(x, wv_ref[...], preferred_element_type=jnp.float32)
    sg_ref[...] = jax.nn.sigmoid(g_ref[...])


def _proj(x2d, Wk, Wv, gates2d, interpret=False):
    return pl.pallas_call(
        _proj_body,
        grid=(D // _BN,),
        in_specs=[
            pl.BlockSpec((B * L, D), lambda j: (0, 0)),
            pl.BlockSpec((D, _BN), lambda j: (0, j)),
            pl.BlockSpec((D, _BN), lambda j: (0, j)),
            pl.BlockSpec((1, E), lambda j: (0, 0)),
        ],
        out_specs=[
            pl.BlockSpec((B * L, _BN), lambda j: (0, j)),
            pl.BlockSpec((B * L, _BN), lambda j: (0, j)),
            pl.BlockSpec((1, E), lambda j: (0, 0)),
        ],
        out_shape=[
            jax.ShapeDtypeStruct((B * L, D), jnp.float32),
            jax.ShapeDtypeStruct((B * L, D), jnp.float32),
            jax.ShapeDtypeStruct((1, E), jnp.float32),
        ],
        interpret=interpret,
    )(x2d, Wk, Wv, gates2d)


_BS = 1024


def _attend_body(tid_ref, sg_ref, aw_ref, v_ref, o_ref):
    b = pl.program_id(0)
    g = sg_ref[0, tid_ref[b]]
    x = aw_ref[0, 0]
    m = jnp.max(x, axis=-1, keepdims=True)
    e = jnp.exp(x - m)
    s = jnp.sum(e, axis=-1, keepdims=True)
    r = e * (g / s)
    v = v_ref[0, 0]
    o_ref[0, 0] = jnp.dot(r, v, preferred_element_type=jnp.float32)


def _attend(aw, v_heads_t, sg, task_ids, interpret=False):
    return pl.pallas_call(
        _attend_body,
        grid=(B, H, S // _BS),
        in_specs=[
            pl.BlockSpec(memory_space=pltpu.SMEM),
            pl.BlockSpec(memory_space=pltpu.SMEM),
            pl.BlockSpec((1, 1, _BS, L), lambda b, h, s: (b, h, s, 0)),
            pl.BlockSpec((1, 1, L, DH), lambda b, h, s: (b, h, 0, 0)),
        ],
        out_specs=pl.BlockSpec((1, 1, _BS, DH), lambda b, h, s: (b, h, s, 0)),
        out_shape=jax.ShapeDtypeStruct((B, H, S, DH), jnp.float32),
        interpret=interpret,
    )(task_ids, sg, aw, v_heads_t)


def kernel(task_ids, expert_prompts, Wk, Wv, gates, adapter_weights):
    task_ids = task_ids.astype(jnp.int32)
    ids16 = jnp.zeros((16,), jnp.int32).at[:B].set(task_ids)
    prompts_flat = _sc_gather(ids16, expert_prompts.reshape(E, L * D))
    x2d = prompts_flat.reshape(B * L, D)
    k2d, v2d, sg = _proj(x2d, Wk, Wv, gates.reshape(1, E))
    adapter_k = k2d.reshape(B, L, D)
    v_heads_t = jnp.transpose(v2d.reshape(B, L, H, DH), (0, 2, 1, 3))
    out = _attend(adapter_weights, v_heads_t, sg, task_ids)
    return out, adapter_k
